# baseline (speedup 1.0000x reference)
import functools

import jax
import jax.numpy as jnp
from jax import lax
from jax.experimental import pallas as pl
from jax.experimental.pallas import tpu as pltpu
from jax.experimental.pallas import tpu_sc as plsc

B = 16384
D = 64
NC = 2
NS = 16
NW = NC * NS
RPW = B // NW
CH = 128
NCH = RPW // CH
G = RPW // 16


def _body(head_h, rel_h, tail_h, ent_h, remb_h, out_h,
          hidx, ridx, tidx, hrow, rrow, trow, outv, sem):
    wid = lax.axis_index("s") * NC + lax.axis_index("c")
    base = wid * RPW

    pltpu.sync_copy(head_h.at[pl.ds(base, RPW)], hidx)
    pltpu.sync_copy(rel_h.at[pl.ds(base, RPW)], ridx)
    pltpu.sync_copy(tail_h.at[pl.ds(base, RPW)], tidx)

    copies = []
    for j in range(NCH):
        sl = pl.ds(j * CH, CH)
        copies.append(pltpu.async_copy(ent_h.at[hidx.at[sl]], hrow.at[sl], sem))
        copies.append(pltpu.async_copy(ent_h.at[tidx.at[sl]], trow.at[sl], sem))
        copies.append(pltpu.async_copy(remb_h.at[ridx.at[sl]], rrow.at[sl], sem))
    for c in copies:
        c.wait()

    iota = lax.iota(jnp.int32, 16)

    def group(g, carry):
        rbase = g * 16
        outvec = jnp.zeros((16,), jnp.float32)
        for i in range(16):
            row = rbase + i
            acc = None
            for c in range(D // 16):
                sl = pl.ds(c * 16, 16)
                hv = hrow[row, sl]
                rv = rrow[row, sl]
                tv = trow[row, sl]
                d = jnp.abs(hv + rv - tv)
                acc = d if acc is None else acc + d
            s = jnp.sum(acc)
            outvec = jnp.where(iota == i, s, outvec)
        outv[pl.ds(rbase, 16)] = 0.0 - outvec
        return carry

    lax.fori_loop(0, G, group, 0)
    pltpu.sync_copy(outv, out_h.at[pl.ds(base, RPW)])


@jax.jit
def _transe_sc(head, rel, tail, ent_embedding, rel_embedding):
    mesh = plsc.VectorSubcoreMesh(core_axis_name="c", subcore_axis_name="s")
    fn = pl.kernel(
        _body,
        out_type=jax.ShapeDtypeStruct((B,), jnp.float32),
        mesh=mesh,
        compiler_params=pltpu.CompilerParams(
            needs_layout_passes=False, use_tc_tiling_on_sc=False),
        scratch_types=[
            pltpu.VMEM((RPW,), jnp.int32),
            pltpu.VMEM((RPW,), jnp.int32),
            pltpu.VMEM((RPW,), jnp.int32),
            pltpu.VMEM((RPW, D), jnp.float32),
            pltpu.VMEM((RPW, D), jnp.float32),
            pltpu.VMEM((RPW, D), jnp.float32),
            pltpu.VMEM((RPW,), jnp.float32),
            pltpu.SemaphoreType.DMA,
        ],
    )
    return fn(head, rel, tail, ent_embedding, rel_embedding)


def kernel(head, rel, tail, ent_embedding, rel_embedding):
    out = _transe_sc(head, rel, tail, ent_embedding, rel_embedding)
    return out.reshape(B, 1)
